# baseline (speedup 1.0000x reference)
import functools

import jax
import jax.numpy as jnp
from jax import lax
from jax.experimental import pallas as pl
from jax.experimental.pallas import tpu as pltpu
from jax.experimental.pallas import tpu_sc as plsc

N_NODES = 10000
NP = 10240
N_EDGES = 320000
D = 128
NC = 2
NS = 16
NW = NC * NS
E_PER_TILE = N_EDGES // NW
NBF = 128
NFULL = E_PER_TILE // NBF
REM = E_PER_TILE - NFULL * NBF
ROWS_PER_TILE = NP // NS
CMARK = 4096.0


def _sc_scatter(edata, recv, zsum):
  mesh = plsc.VectorSubcoreMesh(
      core_axis_name="c", subcore_axis_name="s", num_cores=NC, num_subcores=NS)

  @functools.partial(
      pl.kernel,
      out_type=jax.ShapeDtypeStruct((NC * NP, D), jnp.float32),
      mesh=mesh,
      scratch_types=dict(
          idx_a=pltpu.VMEM((NBF,), jnp.int32),
          idx_b=pltpu.VMEM((NBF,), jnp.int32),
          buf_a=pltpu.VMEM((NBF, D), jnp.float32),
          buf_b=pltpu.VMEM((NBF, D), jnp.float32),
          idx_r=pltpu.VMEM((REM,), jnp.int32),
          buf_r=pltpu.VMEM((REM, D), jnp.float32),
          acc=pltpu.VMEM_SHARED((NP, D), jnp.float32),
          s_ia=pltpu.SemaphoreType.DMA,
          s_ib=pltpu.SemaphoreType.DMA,
          s_ea=pltpu.SemaphoreType.DMA,
          s_eb=pltpu.SemaphoreType.DMA,
          s_sa=pltpu.SemaphoreType.DMA,
          s_sb=pltpu.SemaphoreType.DMA,
      ),
  )
  def k(edata_hbm, recv_hbm, zsum_hbm, out,
        idx_a, idx_b, buf_a, buf_b, idx_r, buf_r, acc,
        s_ia, s_ib, s_ea, s_eb, s_sa, s_sb):
    c = lax.axis_index("c")
    s = lax.axis_index("s")
    wid = c * NS + s
    r0 = s * ROWS_PER_TILE
    out_base = c * NP + r0
    e0 = wid * E_PER_TILE

    def start(k_, idx_v, buf_v, s_i, s_e):
      base = e0 + k_ * NBF
      pltpu.async_copy(recv_hbm.at[pl.ds(base, NBF)], idx_v, s_i)
      pltpu.async_copy(edata_hbm.at[pl.ds(base, NBF)], buf_v, s_e)

    def wait_load(k_, idx_v, buf_v, s_i, s_e):
      base = e0 + k_ * NBF
      pltpu.make_async_copy(recv_hbm.at[pl.ds(base, NBF)], idx_v, s_i).wait()
      pltpu.make_async_copy(edata_hbm.at[pl.ds(base, NBF)], buf_v, s_e).wait()

    cvec = jnp.where(lax.iota(jnp.int32, 16) == 0, CMARK, 0.0).astype(jnp.float32)

    def add_marker(buf_v, n):
      def rb(j, carry):
        for u in range(8):
          r_ = j * 8 + u
          buf_v[r_, pl.ds(0, 16)] = buf_v[r_, pl.ds(0, 16)] + cvec
        return carry

      lax.fori_loop(0, n // 8, rb, 0)

    pltpu.sync_copy(zsum_hbm, acc.at[pl.ds(r0, ROWS_PER_TILE)])
    start(0, idx_a, buf_a, s_ia, s_ea)
    start(1, idx_b, buf_b, s_ib, s_eb)
    plsc.subcore_barrier()

    def body(i, carry):
      ka = 2 * i
      kb = 2 * i + 1
      wait_load(ka, idx_a, buf_a, s_ia, s_ea)
      add_marker(buf_a, NBF)
      pltpu.async_copy(buf_a, acc.at[idx_a], s_sa, add=True)

      wait_load(kb, idx_b, buf_b, s_ib, s_eb)
      add_marker(buf_b, NBF)
      pltpu.async_copy(buf_b, acc.at[idx_b], s_sb, add=True)

      pltpu.make_async_copy(buf_a, acc.at[idx_a], s_sa).wait()

      @pl.when(ka + 2 < NFULL)
      def _():
        start(ka + 2, idx_a, buf_a, s_ia, s_ea)

      pltpu.make_async_copy(buf_b, acc.at[idx_b], s_sb).wait()

      @pl.when(kb + 2 < NFULL)
      def _():
        start(kb + 2, idx_b, buf_b, s_ib, s_eb)

      return carry

    lax.fori_loop(0, NFULL // 2, body, 0)
    base_r = e0 + NFULL * NBF
    pltpu.sync_copy(recv_hbm.at[pl.ds(base_r, REM)], idx_r)
    pltpu.sync_copy(edata_hbm.at[pl.ds(base_r, REM)], buf_r)
    add_marker(buf_r, REM)
    pltpu.sync_copy(buf_r, acc.at[idx_r], add=True)

    plsc.subcore_barrier()
    pltpu.sync_copy(acc.at[pl.ds(r0, ROWS_PER_TILE)],
                    out.at[pl.ds(out_base, ROWS_PER_TILE)])

  return k(edata, recv, zsum)


BM = 2000


def _dense(vdata, W2, b2):
  def body(v_ref, w_ref, b_ref, o_ref):
    o_ref[...] = jnp.dot(v_ref[...], w_ref[...],
                         preferred_element_type=jnp.float32) + b_ref[...]

  return pl.pallas_call(
      body,
      grid=(N_NODES // BM,),
      in_specs=[
          pl.BlockSpec((BM, D), lambda i: (i, 0)),
          pl.BlockSpec((D, D), lambda i: (0, 0)),
          pl.BlockSpec((1, D), lambda i: (0, 0)),
      ],
      out_specs=pl.BlockSpec((BM, D), lambda i: (i, 0)),
      out_shape=jax.ShapeDtypeStruct((N_NODES, D), jnp.float32),
  )(vdata, W2, b2)


def _combine(sums_p, dense, W1):
  def body(s_ref, d_ref, w_ref, o_ref):
    s = s_ref[0] + s_ref[1]
    cnt = jnp.round(s[:, 0:1] * (1.0 / CMARK))
    cntc = jnp.maximum(cnt, 1.0)
    agg0 = (s[:, 0:1] - CMARK * cnt) / cntc
    agg = jnp.concatenate([agg0, s[:, 1:] / cntc], axis=1)
    o_ref[...] = jnp.dot(agg, w_ref[...],
                         preferred_element_type=jnp.float32) + d_ref[...]

  return pl.pallas_call(
      body,
      grid=(N_NODES // BM,),
      in_specs=[
          pl.BlockSpec((NC, BM, D), lambda i: (0, i, 0)),
          pl.BlockSpec((BM, D), lambda i: (i, 0)),
          pl.BlockSpec((D, D), lambda i: (0, 0)),
      ],
      out_specs=pl.BlockSpec((BM, D), lambda i: (i, 0)),
      out_shape=jax.ShapeDtypeStruct((N_NODES, D), jnp.float32),
  )(sums_p, dense, W1)


def kernel(vdata, edata, connectivity, W, b):
  recv = connectivity[1]
  zsum = jnp.zeros((ROWS_PER_TILE, D), jnp.float32)
  acc_p = _sc_scatter(edata, recv, zsum)
  dense = _dense(vdata, W[D:], b.reshape(1, D))
  acc_p = acc_p.reshape(NC, NP, D)
  return _combine(acc_p, dense, W[:D])


# speedup vs baseline: 1.0998x; 1.0998x over previous
import functools

import jax
import jax.numpy as jnp
from jax import lax
from jax.experimental import pallas as pl
from jax.experimental.pallas import tpu as pltpu
from jax.experimental.pallas import tpu_sc as plsc

N_NODES = 10000
NP = 10240
N_EDGES = 320000
D = 128
NC = 2
NS = 16
NW = NC * NS
E_PER_TILE = N_EDGES // NW
NBF = 128
NFULL = E_PER_TILE // NBF
REM = E_PER_TILE - NFULL * NBF
ROWS_PER_TILE = NP // NS
CMARK = 4096.0


def _sc_scatter(edata, recv, zsum):
  mesh = plsc.VectorSubcoreMesh(
      core_axis_name="c", subcore_axis_name="s", num_cores=NC, num_subcores=NS)

  @functools.partial(
      pl.kernel,
      out_type=jax.ShapeDtypeStruct((NC * NP, D), jnp.float32),
      mesh=mesh,
      scratch_types=dict(
          idx_a=pltpu.VMEM((NBF,), jnp.int32),
          idx_b=pltpu.VMEM((NBF,), jnp.int32),
          buf_a=pltpu.VMEM((NBF, D), jnp.float32),
          buf_b=pltpu.VMEM((NBF, D), jnp.float32),
          idx_r=pltpu.VMEM((REM,), jnp.int32),
          buf_r=pltpu.VMEM((REM, D), jnp.float32),
          acc=pltpu.VMEM_SHARED((NP, D), jnp.float32),
          s_ia=pltpu.SemaphoreType.DMA,
          s_ib=pltpu.SemaphoreType.DMA,
          s_ea=pltpu.SemaphoreType.DMA,
          s_eb=pltpu.SemaphoreType.DMA,
          s_sa=pltpu.SemaphoreType.DMA,
          s_sb=pltpu.SemaphoreType.DMA,
      ),
  )
  def k(edata_hbm, recv_hbm, zsum_hbm, out,
        idx_a, idx_b, buf_a, buf_b, idx_r, buf_r, acc,
        s_ia, s_ib, s_ea, s_eb, s_sa, s_sb):
    c = lax.axis_index("c")
    s = lax.axis_index("s")
    wid = c * NS + s
    r0 = s * ROWS_PER_TILE
    out_base = c * NP + r0
    e0 = wid * E_PER_TILE

    def start(k_, idx_v, buf_v, s_i, s_e):
      base = e0 + k_ * NBF
      pltpu.async_copy(recv_hbm.at[pl.ds(base, NBF)], idx_v, s_i)
      pltpu.async_copy(edata_hbm.at[pl.ds(base, NBF)], buf_v, s_e)

    def wait_load(k_, idx_v, buf_v, s_i, s_e):
      base = e0 + k_ * NBF
      pltpu.make_async_copy(recv_hbm.at[pl.ds(base, NBF)], idx_v, s_i).wait()
      pltpu.make_async_copy(edata_hbm.at[pl.ds(base, NBF)], buf_v, s_e).wait()

    cvec = jnp.where(lax.iota(jnp.int32, 16) == 0, CMARK, 0.0).astype(jnp.float32)

    def add_marker(buf_v, n):
      def rb(j, carry):
        buf_v[j, pl.ds(0, 16)] = buf_v[j, pl.ds(0, 16)] + cvec
        return carry

      lax.fori_loop(0, n, rb, 0)

    pltpu.sync_copy(zsum_hbm, acc.at[pl.ds(r0, ROWS_PER_TILE)])
    start(0, idx_a, buf_a, s_ia, s_ea)
    start(1, idx_b, buf_b, s_ib, s_eb)
    plsc.subcore_barrier()

    def body(i, carry):
      ka = 2 * i
      kb = 2 * i + 1
      wait_load(ka, idx_a, buf_a, s_ia, s_ea)
      add_marker(buf_a, NBF)
      pltpu.async_copy(buf_a, acc.at[idx_a], s_sa, add=True)

      wait_load(kb, idx_b, buf_b, s_ib, s_eb)
      add_marker(buf_b, NBF)
      pltpu.async_copy(buf_b, acc.at[idx_b], s_sb, add=True)

      pltpu.make_async_copy(buf_a, acc.at[idx_a], s_sa).wait()

      @pl.when(ka + 2 < NFULL)
      def _():
        start(ka + 2, idx_a, buf_a, s_ia, s_ea)

      pltpu.make_async_copy(buf_b, acc.at[idx_b], s_sb).wait()

      @pl.when(kb + 2 < NFULL)
      def _():
        start(kb + 2, idx_b, buf_b, s_ib, s_eb)

      return carry

    lax.fori_loop(0, NFULL // 2, body, 0)
    base_r = e0 + NFULL * NBF
    pltpu.sync_copy(recv_hbm.at[pl.ds(base_r, REM)], idx_r)
    pltpu.sync_copy(edata_hbm.at[pl.ds(base_r, REM)], buf_r)
    add_marker(buf_r, REM)
    pltpu.sync_copy(buf_r, acc.at[idx_r], add=True)

    plsc.subcore_barrier()
    pltpu.sync_copy(acc.at[pl.ds(r0, ROWS_PER_TILE)],
                    out.at[pl.ds(out_base, ROWS_PER_TILE)])

  return k(edata, recv, zsum)


BM = 2000


def _dense(vdata, W2, b2):
  def body(v_ref, w_ref, b_ref, o_ref):
    o_ref[...] = jnp.dot(v_ref[...], w_ref[...],
                         preferred_element_type=jnp.float32) + b_ref[...]

  return pl.pallas_call(
      body,
      grid=(N_NODES // BM,),
      in_specs=[
          pl.BlockSpec((BM, D), lambda i: (i, 0)),
          pl.BlockSpec((D, D), lambda i: (0, 0)),
          pl.BlockSpec((1, D), lambda i: (0, 0)),
      ],
      out_specs=pl.BlockSpec((BM, D), lambda i: (i, 0)),
      out_shape=jax.ShapeDtypeStruct((N_NODES, D), jnp.float32),
  )(vdata, W2, b2)


def _combine(sums_p, dense, W1):
  def body(s_ref, d_ref, w_ref, o_ref):
    s = s_ref[0] + s_ref[1]
    cnt = jnp.round(s[:, 0:1] * (1.0 / CMARK))
    cntc = jnp.maximum(cnt, 1.0)
    agg0 = (s[:, 0:1] - CMARK * cnt) / cntc
    agg = jnp.concatenate([agg0, s[:, 1:] / cntc], axis=1)
    o_ref[...] = jnp.dot(agg, w_ref[...],
                         preferred_element_type=jnp.float32) + d_ref[...]

  return pl.pallas_call(
      body,
      grid=(N_NODES // BM,),
      in_specs=[
          pl.BlockSpec((NC, BM, D), lambda i: (0, i, 0)),
          pl.BlockSpec((BM, D), lambda i: (i, 0)),
          pl.BlockSpec((D, D), lambda i: (0, 0)),
      ],
      out_specs=pl.BlockSpec((BM, D), lambda i: (i, 0)),
      out_shape=jax.ShapeDtypeStruct((N_NODES, D), jnp.float32),
  )(sums_p, dense, W1)


def kernel(vdata, edata, connectivity, W, b):
  recv = connectivity[1]
  zsum = jnp.zeros((ROWS_PER_TILE, D), jnp.float32)
  acc_p = _sc_scatter(edata, recv, zsum)
  dense = _dense(vdata, W[D:], b.reshape(1, D))
  acc_p = acc_p.reshape(NC, NP, D)
  return _combine(acc_p, dense, W[:D])


# speedup vs baseline: 1.1143x; 1.0132x over previous
import functools

import jax
import jax.numpy as jnp
from jax import lax
from jax.experimental import pallas as pl
from jax.experimental.pallas import tpu as pltpu
from jax.experimental.pallas import tpu_sc as plsc

N_NODES = 10000
NP = 10240
N_EDGES = 320000
D = 128
NC = 2
NS = 16
NW = NC * NS
E_PER_TILE = N_EDGES // NW
NBF = 128
NFULL = E_PER_TILE // NBF
REM = E_PER_TILE - NFULL * NBF
ROWS_PER_TILE = NP // NS
CMARK = 4096.0


def _sc_scatter(edata, recv, zsum):
  mesh = plsc.VectorSubcoreMesh(
      core_axis_name="c", subcore_axis_name="s", num_cores=NC, num_subcores=NS)

  @functools.partial(
      pl.kernel,
      out_type=jax.ShapeDtypeStruct((NC * NP, D), jnp.float32),
      mesh=mesh,
      scratch_types=dict(
          idx_a=pltpu.VMEM((NBF,), jnp.int32),
          idx_b=pltpu.VMEM((NBF,), jnp.int32),
          buf_a=pltpu.VMEM((NBF, D), jnp.float32),
          buf_b=pltpu.VMEM((NBF, D), jnp.float32),
          idx_r=pltpu.VMEM((REM,), jnp.int32),
          buf_r=pltpu.VMEM((REM, D), jnp.float32),
          acc=pltpu.VMEM_SHARED((NP, D), jnp.float32),
          s_ia=pltpu.SemaphoreType.DMA,
          s_ib=pltpu.SemaphoreType.DMA,
          s_ea=pltpu.SemaphoreType.DMA,
          s_eb=pltpu.SemaphoreType.DMA,
          s_sa=pltpu.SemaphoreType.DMA,
          s_sb=pltpu.SemaphoreType.DMA,
      ),
  )
  def k(edata_hbm, recv_hbm, zsum_hbm, out,
        idx_a, idx_b, buf_a, buf_b, idx_r, buf_r, acc,
        s_ia, s_ib, s_ea, s_eb, s_sa, s_sb):
    c = lax.axis_index("c")
    s = lax.axis_index("s")
    wid = c * NS + s
    r0 = s * ROWS_PER_TILE
    out_base = c * NP + r0
    e0 = wid * E_PER_TILE

    def start(k_, idx_v, buf_v, s_i, s_e):
      base = e0 + k_ * NBF
      pltpu.async_copy(recv_hbm.at[pl.ds(base, NBF)], idx_v, s_i)
      pltpu.async_copy(edata_hbm.at[pl.ds(base, NBF)], buf_v, s_e)

    def wait_load(k_, idx_v, buf_v, s_i, s_e):
      base = e0 + k_ * NBF
      pltpu.make_async_copy(recv_hbm.at[pl.ds(base, NBF)], idx_v, s_i).wait()
      pltpu.make_async_copy(edata_hbm.at[pl.ds(base, NBF)], buf_v, s_e).wait()

    cvec = jnp.where(lax.iota(jnp.int32, 16) == 0, CMARK, 0.0).astype(jnp.float32)

    def add_marker(buf_v, n):
      def rb(j, carry):
        buf_v[j, pl.ds(0, 16)] = buf_v[j, pl.ds(0, 16)] + cvec
        return carry

      lax.fori_loop(0, n, rb, 0)

    pltpu.sync_copy(zsum_hbm, acc.at[pl.ds(r0, ROWS_PER_TILE)])
    start(0, idx_a, buf_a, s_ia, s_ea)
    start(1, idx_b, buf_b, s_ib, s_eb)
    plsc.subcore_barrier()

    def body(i, carry):
      ka = 2 * i
      kb = 2 * i + 1
      wait_load(ka, idx_a, buf_a, s_ia, s_ea)
      add_marker(buf_a, NBF)
      pltpu.async_copy(buf_a, acc.at[idx_a], s_sa, add=True)

      wait_load(kb, idx_b, buf_b, s_ib, s_eb)
      add_marker(buf_b, NBF)
      pltpu.async_copy(buf_b, acc.at[idx_b], s_sb, add=True)

      pltpu.make_async_copy(buf_a, acc.at[idx_a], s_sa).wait()

      @pl.when(ka + 2 < NFULL)
      def _():
        start(ka + 2, idx_a, buf_a, s_ia, s_ea)

      pltpu.make_async_copy(buf_b, acc.at[idx_b], s_sb).wait()

      @pl.when(kb + 2 < NFULL)
      def _():
        start(kb + 2, idx_b, buf_b, s_ib, s_eb)

      return carry

    lax.fori_loop(0, NFULL // 2, body, 0)
    base_r = e0 + NFULL * NBF
    pltpu.sync_copy(recv_hbm.at[pl.ds(base_r, REM)], idx_r)
    pltpu.sync_copy(edata_hbm.at[pl.ds(base_r, REM)], buf_r)
    add_marker(buf_r, REM)
    pltpu.sync_copy(buf_r, acc.at[idx_r], add=True)

    plsc.subcore_barrier()
    pltpu.sync_copy(acc.at[pl.ds(r0, ROWS_PER_TILE)],
                    out.at[pl.ds(out_base, ROWS_PER_TILE)])

  return k(edata, recv, zsum)


BM = 2000


def _combine(sums_p, vdata, W, b2):
  def body(s_ref, v_ref, w_ref, b_ref, o_ref):
    s = s_ref[0] + s_ref[1]
    cnt = jnp.round(s[:, 0:1] * (1.0 / CMARK))
    cntc = jnp.maximum(cnt, 1.0)
    agg0 = (s[:, 0:1] - CMARK * cnt) / cntc
    agg = jnp.concatenate([agg0, s[:, 1:] / cntc], axis=1)
    o_ref[...] = (
        jnp.dot(agg, w_ref[0:D, :], preferred_element_type=jnp.float32)
        + jnp.dot(v_ref[...], w_ref[D:2 * D, :], preferred_element_type=jnp.float32)
        + b_ref[...]
    )

  return pl.pallas_call(
      body,
      grid=(N_NODES // BM,),
      in_specs=[
          pl.BlockSpec((NC, BM, D), lambda i: (0, i, 0)),
          pl.BlockSpec((BM, D), lambda i: (i, 0)),
          pl.BlockSpec((2 * D, D), lambda i: (0, 0)),
          pl.BlockSpec((1, D), lambda i: (0, 0)),
      ],
      out_specs=pl.BlockSpec((BM, D), lambda i: (i, 0)),
      out_shape=jax.ShapeDtypeStruct((N_NODES, D), jnp.float32),
  )(sums_p, vdata, W, b2)


def kernel(vdata, edata, connectivity, W, b):
  recv = connectivity[1]
  zsum = jnp.zeros((ROWS_PER_TILE, D), jnp.float32)
  acc_p = _sc_scatter(edata, recv, zsum)
  acc_p = acc_p.reshape(NC, NP, D)
  return _combine(acc_p, vdata, W, b.reshape(1, D))
